# baseline (speedup 1.0000x reference)
import functools

import jax
import jax.numpy as jnp
from jax import lax
from jax.experimental import pallas as pl
from jax.experimental.pallas import tpu as pltpu
from jax.experimental.pallas import tpu_sc as plsc

_BM = 1024


def _make_sc_gather(n_idx: int):
    info = plsc.get_sparse_core_info()
    nc, ns, lanes = info.num_cores, info.num_subcores, info.num_lanes
    nw = nc * ns
    chunk = n_idx // nw
    assert chunk % lanes == 0 and chunk % 8 == 0

    mesh = plsc.VectorSubcoreMesh(core_axis_name="c", subcore_axis_name="s")

    @functools.partial(
        pl.kernel,
        out_type=jax.ShapeDtypeStruct((n_idx,), jnp.int32),
        mesh=mesh,
        scratch_types=[
            pltpu.VMEM((chunk,), jnp.int32),
            pltpu.VMEM((chunk,), jnp.int32),
            pltpu.SemaphoreType.DMA,
        ],
    )
    def sc_gather(idx_hbm, doc_hbm, out_hbm, idx_v, out_v, sem):
        wid = lax.axis_index("s") * nc + lax.axis_index("c")
        base = wid * chunk
        pltpu.sync_copy(idx_hbm.at[pl.ds(base, chunk)], idx_v)
        pltpu.async_copy(doc_hbm.at[idx_v], out_v, sem).wait()
        pltpu.sync_copy(out_v, out_hbm.at[pl.ds(base, chunk)])

    return sc_gather


def _tc_cmp_body(dq_ref, dk_ref, out_ref):
    out_ref[...] = (dq_ref[...] == dk_ref[...]).astype(jnp.int8)


def _tc_compare(dq, dk):
    s = dk.shape[1]
    return pl.pallas_call(
        _tc_cmp_body,
        grid=(dq.shape[0] // _BM,),
        in_specs=[
            pl.BlockSpec((_BM, 1), lambda i: (i, 0)),
            pl.BlockSpec((1, s), lambda i: (0, 0)),
        ],
        out_specs=pl.BlockSpec((_BM, s), lambda i: (i, 0)),
        out_shape=jax.ShapeDtypeStruct((dq.shape[0], s), jnp.int8),
    )(dq, dk)


def kernel(b, h, q, kv, doc_ids):
    s = doc_ids.shape[0]
    idx = jnp.concatenate([q.reshape(-1), kv.reshape(-1)])
    gathered = _make_sc_gather(2 * s)(idx, doc_ids).astype(jnp.int8)
    dq = gathered[:s].reshape(s, 1)
    dk = gathered[s:].reshape(1, s)
    return _tc_compare(dq, dk).astype(jnp.bool_)
